# Initial kernel scaffold; baseline (speedup 1.0000x reference)
import jax
import jax.numpy as jnp
from jax.experimental import pallas as pl


def kernel(idx, logits):
    raise NotImplementedError("write your pallas kernel here")


# speedup vs baseline: 1.7358x; 1.7358x over previous
import functools

import jax
import jax.numpy as jnp
from jax import lax
from jax.experimental import pallas as pl
from jax.experimental.pallas import tpu as pltpu
from jax.experimental.pallas import tpu_sc as plsc

V = 8192
D = 8192
B = 4096

_info = plsc.get_sparse_core_info()
_NC, _NS = _info.num_cores, _info.num_subcores
NW = _NC * _NS
B_PER_W = B // NW
K = 8
NCH = B_PER_W // K

_mesh = plsc.VectorSubcoreMesh(core_axis_name="c", subcore_axis_name="s")


@functools.partial(
    pl.kernel,
    mesh=_mesh,
    out_type=jax.ShapeDtypeStruct((B, D), jnp.float32),
    scratch_types=[
        pltpu.VMEM((B_PER_W,), jnp.int32),
        pltpu.VMEM((K, D), jnp.float32),
        pltpu.SemaphoreType.DMA,
    ],
)
def _gather_rows(table_hbm, idx_hbm, out_hbm, idx_v, rows_v, sem):
    wid = lax.axis_index("s") * _NC + lax.axis_index("c")
    base = wid * B_PER_W
    pltpu.sync_copy(idx_hbm.at[pl.ds(base, B_PER_W)], idx_v)

    def chunk(g, carry):
        pltpu.async_copy(
            table_hbm.at[idx_v.at[pl.ds(g * K, K)]], rows_v, sem
        ).wait()
        pltpu.sync_copy(rows_v, out_hbm.at[pl.ds(base + g * K, K)])
        return carry

    lax.fori_loop(0, NCH, chunk, 0)


def kernel(idx, logits):
    return _gather_rows(logits, idx)


# speedup vs baseline: 1.8015x; 1.0379x over previous
import functools

import jax
import jax.numpy as jnp
from jax import lax
from jax.experimental import pallas as pl
from jax.experimental.pallas import tpu as pltpu
from jax.experimental.pallas import tpu_sc as plsc

V = 8192
D = 8192
B = 4096

_info = plsc.get_sparse_core_info()
_NC, _NS = _info.num_cores, _info.num_subcores
NW = _NC * _NS
B_PER_W = B // NW
K = 4
NCH = B_PER_W // K
R = NCH // 2

_mesh = plsc.VectorSubcoreMesh(core_axis_name="c", subcore_axis_name="s")


@functools.partial(
    pl.kernel,
    mesh=_mesh,
    out_type=jax.ShapeDtypeStruct((B, D), jnp.float32),
    scratch_types=[
        pltpu.VMEM((NCH, K), jnp.int32),
        pltpu.VMEM((K, D), jnp.float32),
        pltpu.VMEM((K, D), jnp.float32),
        pltpu.SemaphoreType.DMA,
        pltpu.SemaphoreType.DMA,
        pltpu.SemaphoreType.DMA,
        pltpu.SemaphoreType.DMA,
    ],
)
def _gather_rows(table, idx_hbm, out, idx_v, buf0, buf1, gs0, gs1, ws0, ws1):
    wid = lax.axis_index("s") * _NC + lax.axis_index("c")
    base = wid * B_PER_W
    pltpu.sync_copy(idx_hbm.at[wid], idx_v)

    g0 = pltpu.async_copy(table.at[idx_v.at[0]], buf0, gs0)
    g1 = pltpu.async_copy(table.at[idx_v.at[1]], buf1, gs1)
    g0.wait()
    pltpu.async_copy(buf0, out.at[pl.ds(base, K)], ws0)
    g1.wait()
    pltpu.async_copy(buf1, out.at[pl.ds(base + K, K)], ws1)

    def round_body(r, carry):
        c0 = 2 * r
        pltpu.make_async_copy(
            buf0, out.at[pl.ds(base + (c0 - 2) * K, K)], ws0
        ).wait()
        ga = pltpu.async_copy(table.at[idx_v.at[c0]], buf0, gs0)
        pltpu.make_async_copy(
            buf1, out.at[pl.ds(base + (c0 - 1) * K, K)], ws1
        ).wait()
        gb = pltpu.async_copy(table.at[idx_v.at[c0 + 1]], buf1, gs1)
        ga.wait()
        pltpu.async_copy(buf0, out.at[pl.ds(base + c0 * K, K)], ws0)
        gb.wait()
        pltpu.async_copy(buf1, out.at[pl.ds(base + (c0 + 1) * K, K)], ws1)
        return carry

    lax.fori_loop(1, R, round_body, 0)

    pltpu.make_async_copy(
        buf0, out.at[pl.ds(base + (NCH - 2) * K, K)], ws0
    ).wait()
    pltpu.make_async_copy(
        buf1, out.at[pl.ds(base + (NCH - 1) * K, K)], ws1
    ).wait()


def kernel(idx, logits):
    idx3 = idx.astype(jnp.int32).reshape(NW, NCH, K)
    return _gather_rows(logits, idx3)
